# Initial kernel scaffold; baseline (speedup 1.0000x reference)
import jax
import jax.numpy as jnp
from jax.experimental import pallas as pl


def kernel(word_ids, age_ids, bmi_ids, cycle_len_ids, seg_ids, posi_ids, word_table, demo_table, posi_table, seg_table, ln_gamma, ln_beta):
    raise NotImplementedError("write your pallas kernel here")


# speedup vs baseline: 4.6140x; 4.6140x over previous
import dataclasses
import functools

import jax
import jax.numpy as jnp
from jax import lax
from jax.experimental import pallas as pl
from jax.experimental.pallas import tpu as pltpu
from jax.experimental.pallas import tpu_sc as plsc

NC = 2
NS = 16
NW = NC * NS
L16 = 16

HID = 64
KV = HID // L16

DEMO_VOCAB = 128
MAX_POS = 512
SEG_VOCAB = 2

C = 128


def _rsqrt(x):
    i = lax.bitcast_convert_type(x, jnp.int32)
    i = jnp.int32(0x5F3759DF) - lax.shift_right_arithmetic(i, 1)
    y = lax.bitcast_convert_type(i, jnp.float32)
    for _ in range(3):
        y = y * (1.5 - 0.5 * x * y * y)
    return y


@functools.partial(jax.jit, static_argnames=("n_tok",))
def _embed_ln(n_tok, wids, aids, bids, cids, sids, pids,
              wtab, dtab_f, ptab_f, stab_f, gamma, beta):
    tok_w = n_tok // NW
    nchunk = tok_w // C
    mesh = plsc.VectorSubcoreMesh(core_axis_name="c", subcore_axis_name="s")
    cp = pltpu.CompilerParams()
    if "needs_layout_passes" in pltpu.CompilerParams.__dataclass_fields__:
        cp = dataclasses.replace(cp, needs_layout_passes=False)
    if "use_tc_tiling_on_sc" in pltpu.CompilerParams.__dataclass_fields__:
        cp = dataclasses.replace(cp, use_tc_tiling_on_sc=False)

    @functools.partial(
        pl.kernel,
        compiler_params=cp,
        out_type=jax.ShapeDtypeStruct((n_tok, HID), jnp.float32),
        mesh=mesh,
        scratch_types=[
            pltpu.VMEM((C,), jnp.int32),
            pltpu.VMEM((C,), jnp.int32),
            pltpu.VMEM((C,), jnp.int32),
            pltpu.VMEM((C,), jnp.int32),
            pltpu.VMEM((C,), jnp.int32),
            pltpu.VMEM((C,), jnp.int32),
            pltpu.VMEM((C, HID), jnp.float32),
            pltpu.VMEM((C, HID), jnp.float32),
            pltpu.VMEM((DEMO_VOCAB * HID,), jnp.float32),
            pltpu.VMEM((MAX_POS * HID,), jnp.float32),
            pltpu.VMEM((SEG_VOCAB * HID,), jnp.float32),
            pltpu.VMEM((HID,), jnp.float32),
            pltpu.VMEM((HID,), jnp.float32),
        ],
    )
    def k(wids_h, aids_h, bids_h, cids_h, sids_h, pids_h,
          wtab_h, dtab_h, ptab_h, stab_h, gamma_h, beta_h, out_h,
          widx, aidx, bidx, cidx, sidx, pidx, wrows, obuf,
          dtab_v, ptab_v, stab_v, g_v, b_v):
        wid = lax.axis_index("s") * NC + lax.axis_index("c")
        base = wid * tok_w

        pltpu.sync_copy(dtab_h, dtab_v)
        pltpu.sync_copy(ptab_h, ptab_v)
        pltpu.sync_copy(stab_h, stab_v)
        pltpu.sync_copy(gamma_h, g_v)
        pltpu.sync_copy(beta_h, b_v)

        @pl.loop(0, nchunk)
        def _chunk(g):
            off = base + g * C
            pltpu.sync_copy(wids_h.at[pl.ds(off, C)], widx)
            pltpu.sync_copy(aids_h.at[pl.ds(off, C)], aidx)
            pltpu.sync_copy(bids_h.at[pl.ds(off, C)], bidx)
            pltpu.sync_copy(cids_h.at[pl.ds(off, C)], cidx)
            pltpu.sync_copy(sids_h.at[pl.ds(off, C)], sidx)
            pltpu.sync_copy(pids_h.at[pl.ds(off, C)], pidx)
            pltpu.sync_copy(wtab_h.at[widx], wrows)

            @pl.loop(0, C // L16)
            def _grp(gg):
                s = gg * L16
                aidv = aidx[pl.ds(s, L16)] * HID
                bidv = bidx[pl.ds(s, L16)] * HID
                cidv = cidx[pl.ds(s, L16)] * HID
                sidv = sidx[pl.ds(s, L16)] * HID
                pidv = pidx[pl.ds(s, L16)] * HID
                gvec = [g_v[pl.ds(kk * L16, L16)] for kk in range(KV)]
                bvec = [b_v[pl.ds(kk * L16, L16)] for kk in range(KV)]

                for j in range(L16):
                    t = s + j
                    aid = aidv[j]
                    bid = bidv[j]
                    cid = cidv[j]
                    sid = sidv[j]
                    pid = pidv[j]

                    acc = []
                    for kk in range(KV):
                        o = kk * L16
                        v = (wrows[t, pl.ds(o, L16)]
                             + dtab_v[pl.ds(aid + o, L16)]
                             + dtab_v[pl.ds(bid + o, L16)]
                             + dtab_v[pl.ds(cid + o, L16)]
                             + ptab_v[pl.ds(pid + o, L16)]
                             + stab_v[pl.ds(sid + o, L16)])
                        acc.append(v)

                    tot = acc[0] + acc[1] + acc[2] + acc[3]
                    mean = jnp.sum(tot) * (1.0 / HID)
                    mvec = jnp.full((L16,), mean, dtype=jnp.float32)
                    d = [a - mvec for a in acc]
                    sq = d[0] * d[0] + d[1] * d[1] + d[2] * d[2] + d[3] * d[3]
                    var = jnp.sum(sq) * (1.0 / HID)
                    rvec = _rsqrt(
                        jnp.full((L16,), var + 1e-12, dtype=jnp.float32))
                    for kk in range(KV):
                        o = kk * L16
                        obuf[t, pl.ds(o, L16)] = (
                            d[kk] * rvec * gvec[kk] + bvec[kk])

            pltpu.sync_copy(obuf, out_h.at[pl.ds(off, C)])

    return k(wids, aids, bids, cids, sids, pids,
             wtab, dtab_f, ptab_f, stab_f, gamma, beta)


def kernel(word_ids, age_ids, bmi_ids, cycle_len_ids, seg_ids, posi_ids,
           word_table, demo_table, posi_table, seg_table, ln_gamma, ln_beta):
    b, l = word_ids.shape
    n_tok = b * l
    as_i32 = lambda x: x.reshape(-1).astype(jnp.int32)
    out = _embed_ln(
        n_tok,
        as_i32(word_ids), as_i32(age_ids), as_i32(bmi_ids),
        as_i32(cycle_len_ids), as_i32(seg_ids), as_i32(posi_ids),
        word_table.astype(jnp.float32),
        demo_table.astype(jnp.float32).reshape(-1),
        posi_table.astype(jnp.float32).reshape(-1),
        seg_table.astype(jnp.float32).reshape(-1),
        ln_gamma.astype(jnp.float32), ln_beta.astype(jnp.float32),
    )
    return out.reshape(b, l, HID)
